# Initial kernel scaffold; baseline (speedup 1.0000x reference)
import jax
import jax.numpy as jnp
from jax import lax
from jax.experimental import pallas as pl
from jax.experimental.pallas import tpu as pltpu
from jax.experimental.pallas import tpu_sc as plsc

BATCH = 16384
N_HIST = 50
D = 32
NW = 32
B_PER_W = BATCH // NW
C = 32
NCH = B_PER_W // C


def _body(idx_hbm, uidx_hbm, utab_hbm, itab_hbm, w_hbm, out_hbm,
          idx_v, uidx_v, rows_v, urows_v, out_v, w_v, gsem, usem):
    nc = plsc.get_sparse_core_info().num_cores
    wid = lax.axis_index("s") * nc + lax.axis_index("c")

    pltpu.sync_copy(w_hbm, w_v)

    def chunk_body(ch, _):
        pltpu.sync_copy(idx_hbm.at[wid, ch], idx_v)
        pltpu.sync_copy(uidx_hbm.at[wid, ch], uidx_v)
        descs = [
            pltpu.async_copy(itab_hbm.at[idx_v.at[n]], rows_v.at[n], gsem)
            for n in range(N_HIST)
        ]
        udesc = pltpu.async_copy(utab_hbm.at[uidx_v], urows_v, usem)
        for dsc in descs:
            dsc.wait()
        udesc.wait()

        bias = w_v[N_HIST]

        def row_body(c, _):
            def n_body(n, acc):
                a0, a1 = acc
                w = w_v[n]
                r0 = rows_v[n, c, pl.ds(0, 16)]
                r1 = rows_v[n, c, pl.ds(16, 16)]
                return (a0 + w * r0, a1 + w * r1)

            z = jnp.full((16,), bias, dtype=jnp.float32)
            t0, t1 = lax.fori_loop(0, N_HIST, n_body, (z, z))
            u0 = urows_v[c, pl.ds(0, 16)]
            u1 = urows_v[c, pl.ds(16, 16)]
            out_v[c, pl.ds(0, 16)] = u0
            out_v[c, pl.ds(16, 16)] = u1
            out_v[c, pl.ds(32, 16)] = u0 * t0
            out_v[c, pl.ds(48, 16)] = u1 * t1
            out_v[c, pl.ds(64, 16)] = t0
            out_v[c, pl.ds(80, 16)] = t1
            return 0

        lax.fori_loop(0, C, row_body, 0)
        pltpu.sync_copy(out_v, out_hbm.at[pl.ds(wid * B_PER_W + ch * C, C)])
        return 0

    lax.fori_loop(0, NCH, chunk_body, 0)


@jax.jit
def _run(idx_r, uidx_r, user_table, item_table, wb):
    mesh = plsc.VectorSubcoreMesh(core_axis_name="c", subcore_axis_name="s")
    f = pl.kernel(
        _body,
        out_type=jax.ShapeDtypeStruct((BATCH, 96), jnp.float32),
        mesh=mesh,
        scratch_types=[
            pltpu.VMEM((N_HIST, C), jnp.int32),
            pltpu.VMEM((C,), jnp.int32),
            pltpu.VMEM((N_HIST, C, D), jnp.float32),
            pltpu.VMEM((C, D), jnp.float32),
            pltpu.VMEM((C, 96), jnp.float32),
            pltpu.VMEM((64,), jnp.float32),
            pltpu.SemaphoreType.DMA,
            pltpu.SemaphoreType.DMA,
        ],
    )
    return f(idx_r, uidx_r, user_table, item_table, wb)


def kernel(user, memory, user_table, item_table, conv_w, conv_b):
    idx_r = memory.reshape(NW, NCH, C, N_HIST).transpose(0, 1, 3, 2)
    uidx_r = user.reshape(NW, NCH, C)
    wb = jnp.concatenate(
        [conv_w, conv_b, jnp.zeros((64 - N_HIST - 1,), jnp.float32)])
    return _run(idx_r, uidx_r, user_table, item_table, wb)


# speedup vs baseline: 5.3718x; 5.3718x over previous
import jax
import jax.numpy as jnp
from jax import lax
from jax.experimental import pallas as pl
from jax.experimental.pallas import tpu as pltpu
from jax.experimental.pallas import tpu_sc as plsc

BATCH = 16384
N_HIST = 50
D = 32
NW = 32
B_PER_W = BATCH // NW
C = 32
NCH = B_PER_W // C


def _body(idx_hbm, uidx_hbm, utab_hbm, itab_hbm, w_hbm, out_hbm,
          idx_v, uidx_v, rows_v, urows_v, out_v, w_v, gsem, usem):
    wid = lax.axis_index("s") * 2 + lax.axis_index("c")

    pltpu.sync_copy(w_hbm, w_v)

    def chunk_body(ch, _):
        pltpu.sync_copy(idx_hbm.at[wid, ch], idx_v)
        pltpu.sync_copy(uidx_hbm.at[wid, ch], uidx_v)
        descs = [
            pltpu.async_copy(itab_hbm.at[idx_v.at[n]], rows_v.at[n], gsem)
            for n in range(N_HIST)
        ]
        udesc = pltpu.async_copy(utab_hbm.at[uidx_v], urows_v, usem)
        for dsc in descs:
            dsc.wait()
        udesc.wait()

        bias = w_v[pl.ds(N_HIST, 16)][0]

        def row_body(c, _):
            def n_body(n, acc):
                a0, a1 = acc
                w = w_v[pl.ds(n, 16)][0]
                r0 = rows_v[n, c, pl.ds(0, 16)]
                r1 = rows_v[n, c, pl.ds(16, 16)]
                return (a0 + w * r0, a1 + w * r1)

            z = jnp.full((16,), bias, dtype=jnp.float32)
            t0, t1 = lax.fori_loop(0, N_HIST, n_body, (z, z))
            u0 = urows_v[c, pl.ds(0, 16)]
            u1 = urows_v[c, pl.ds(16, 16)]
            out_v[c, pl.ds(0, 16)] = u0
            out_v[c, pl.ds(16, 16)] = u1
            out_v[c, pl.ds(32, 16)] = u0 * t0
            out_v[c, pl.ds(48, 16)] = u1 * t1
            out_v[c, pl.ds(64, 16)] = t0
            out_v[c, pl.ds(80, 16)] = t1
            return 0

        lax.fori_loop(0, C, row_body, 0)
        pltpu.sync_copy(out_v, out_hbm.at[pl.ds(wid * B_PER_W + ch * C, C)])
        return 0

    lax.fori_loop(0, NCH, chunk_body, 0)


@jax.jit
def _run(idx_r, uidx_r, user_table, item_table, wb):
    mesh = plsc.VectorSubcoreMesh(
        core_axis_name="c", subcore_axis_name="s", num_cores=2, num_subcores=16)
    f = pl.kernel(
        _body,
        out_type=jax.ShapeDtypeStruct((BATCH, 96), jnp.float32),
        mesh=mesh,
        scratch_types=[
            pltpu.VMEM((N_HIST, C), jnp.int32),
            pltpu.VMEM((C,), jnp.int32),
            pltpu.VMEM((N_HIST, C, D), jnp.float32),
            pltpu.VMEM((C, D), jnp.float32),
            pltpu.VMEM((C, 96), jnp.float32),
            pltpu.VMEM((80,), jnp.float32),
            pltpu.SemaphoreType.DMA,
            pltpu.SemaphoreType.DMA,
        ],
        compiler_params=pltpu.CompilerParams(use_tc_tiling_on_sc=False),
    )
    return f(idx_r, uidx_r, user_table, item_table, wb)


def kernel(user, memory, user_table, item_table, conv_w, conv_b):
    idx_r = memory.reshape(NW, NCH, C, N_HIST).transpose(0, 1, 3, 2)
    uidx_r = user.reshape(NW, NCH, C)
    wb = jnp.concatenate(
        [conv_w, conv_b, jnp.zeros((80 - N_HIST - 1,), jnp.float32)])
    return _run(idx_r, uidx_r, user_table, item_table, wb)


# speedup vs baseline: 6.1633x; 1.1473x over previous
import jax
import jax.numpy as jnp
from jax import lax
from jax.experimental import pallas as pl
from jax.experimental.pallas import tpu as pltpu
from jax.experimental.pallas import tpu_sc as plsc

BATCH = 16384
N_HIST = 50
D = 32
NW = 32
B_PER_W = BATCH // NW
C = 32
NCH = B_PER_W // C
NROW = N_HIST * C
GI = 128
NG = (NROW + GI - 1) // GI


def _body(user_hbm, mem_hbm, utab_hbm, itab_hbm, w_hbm, b_hbm, out_hbm,
          idxr_v, idx_v, uidx_v, rows_v, urows_v, out_v, w_v,
          gsemA, gsemB, osem):
    wid = lax.axis_index("s") * 2 + lax.axis_index("c")
    base = wid * B_PER_W

    pltpu.sync_copy(w_hbm, w_v.at[pl.ds(0, N_HIST)])
    pltpu.sync_copy(b_hbm, w_v.at[pl.ds(56, 1)])

    rows_lo = lax.iota(jnp.int32, 16)
    rows_hi = rows_lo + 16

    def stage(ch, buf, gsem):
        r0 = base + ch * C
        pltpu.sync_copy(mem_hbm.at[pl.ds(r0, C)], idxr_v.at[buf])
        pltpu.sync_copy(user_hbm.at[pl.ds(r0, C)], uidx_v.at[buf])

        def tr_body(n, _):
            cols = jnp.full((16,), n, dtype=jnp.int32)
            g0 = plsc.load_gather(idxr_v.at[buf], [rows_lo, cols])
            g1 = plsc.load_gather(idxr_v.at[buf], [rows_hi, cols])
            idx_v[buf, pl.ds(n * C, 16)] = g0
            idx_v[buf, pl.ds(n * C + 16, 16)] = g1
            return 0

        lax.fori_loop(0, N_HIST, tr_body, 0)

        descs = []
        for j in range(NG):
            lo = j * GI
            sz = min(GI, NROW - lo)
            descs.append(pltpu.async_copy(
                itab_hbm.at[idx_v.at[buf, pl.ds(lo, sz)]],
                rows_v.at[buf, pl.ds(lo, sz)], gsem))
        descs.append(pltpu.async_copy(
            utab_hbm.at[uidx_v.at[buf]], urows_v.at[buf], gsem))
        return descs

    wv = [w_v[pl.ds(k, 16)] for k in (0, 16, 32, 48)]
    bias = w_v[pl.ds(56, 16)][0]

    def compute(ch, buf):
        r0 = base + ch * C

        def row_body(c, _):
            z = jnp.full((16,), bias, dtype=jnp.float32)
            a0 = z
            a1 = z
            for n in range(N_HIST):
                w = wv[n // 16][n % 16]
                fr = n * C + c
                r0v = rows_v[buf, fr, pl.ds(0, 16)]
                r1v = rows_v[buf, fr, pl.ds(16, 16)]
                a0 = a0 + w * r0v
                a1 = a1 + w * r1v
            u0 = urows_v[buf, c, pl.ds(0, 16)]
            u1 = urows_v[buf, c, pl.ds(16, 16)]
            out_v[buf, c, pl.ds(0, 16)] = u0
            out_v[buf, c, pl.ds(16, 16)] = u1
            out_v[buf, c, pl.ds(32, 16)] = u0 * a0
            out_v[buf, c, pl.ds(48, 16)] = u1 * a1
            out_v[buf, c, pl.ds(64, 16)] = a0
            out_v[buf, c, pl.ds(80, 16)] = a1
            return 0

        lax.fori_loop(0, C, row_body, 0)
        return pltpu.async_copy(
            out_v.at[buf], out_hbm.at[pl.ds(r0, C)], osem)

    gsems = (gsemA, gsemB)
    pending = stage(0, 0, gsems[0])
    out_descs = []
    for ch in range(NCH):
        buf = ch % 2
        nxt = None
        if ch + 1 < NCH:
            nxt = stage(ch + 1, 1 - buf, gsems[1 - buf])
        for dsc in pending:
            dsc.wait()
        pending = nxt
        if ch >= 2:
            out_descs[ch - 2].wait()
        out_descs.append(compute(ch, buf))
    out_descs[-2].wait()
    out_descs[-1].wait()


@jax.jit
def _run(user, memory, user_table, item_table, conv_w, conv_b):
    mesh = plsc.VectorSubcoreMesh(
        core_axis_name="c", subcore_axis_name="s", num_cores=2, num_subcores=16)
    f = pl.kernel(
        _body,
        out_type=jax.ShapeDtypeStruct((BATCH, 96), jnp.float32),
        mesh=mesh,
        scratch_types=[
            pltpu.VMEM((2, C, N_HIST), jnp.int32),
            pltpu.VMEM((2, NROW), jnp.int32),
            pltpu.VMEM((2, C), jnp.int32),
            pltpu.VMEM((2, NROW, D), jnp.float32),
            pltpu.VMEM((2, C, D), jnp.float32),
            pltpu.VMEM((2, C, 96), jnp.float32),
            pltpu.VMEM((80,), jnp.float32),
            pltpu.SemaphoreType.DMA,
            pltpu.SemaphoreType.DMA,
            pltpu.SemaphoreType.DMA,
        ],
        compiler_params=pltpu.CompilerParams(use_tc_tiling_on_sc=False, needs_layout_passes=False),
    )
    return f(user, memory, user_table, item_table, conv_w, conv_b)


def kernel(user, memory, user_table, item_table, conv_w, conv_b):
    return _run(user, memory, user_table, item_table, conv_w, conv_b)


# speedup vs baseline: 19.5369x; 3.1698x over previous
import jax
import jax.numpy as jnp
from jax import lax
from jax.experimental import pallas as pl
from jax.experimental.pallas import tpu as pltpu
from jax.experimental.pallas import tpu_sc as plsc

BATCH = 16384
N_HIST = 50
D = 32
NW = 32
B_PER_W = BATCH // NW
C = 32
NCH = B_PER_W // C
NROW = N_HIST * C
GI = 128
NG = (NROW + GI - 1) // GI


def _body(urows_hbm, mem_hbm, itab_hbm, w_hbm, b_hbm, out_hbm,
          idxr_v, idx_v, rows_v, urows_v, out_v, w_v,
          gsemA, gsemB, osem):
    wid = lax.axis_index("s") * 2 + lax.axis_index("c")
    base = wid * B_PER_W

    pltpu.sync_copy(w_hbm, w_v.at[pl.ds(0, N_HIST)])
    pltpu.sync_copy(b_hbm, w_v.at[pl.ds(56, 1)])

    rows_lo = lax.iota(jnp.int32, 16)
    rows_hi = rows_lo + 16

    def stage(ch, buf, gsem):
        r0 = base + ch * C
        pltpu.sync_copy(mem_hbm.at[pl.ds(r0 * N_HIST, C * N_HIST)],
                        idxr_v.at[buf])
        pltpu.sync_copy(urows_hbm.at[pl.ds(r0, C)], urows_v.at[buf])

        def tr_body(n, _):
            g0 = plsc.load_gather(idxr_v.at[buf], [rows_lo * N_HIST + n])
            g1 = plsc.load_gather(idxr_v.at[buf], [rows_hi * N_HIST + n])
            idx_v[buf, pl.ds(n * C, 16)] = g0
            idx_v[buf, pl.ds(n * C + 16, 16)] = g1
            return 0

        lax.fori_loop(0, N_HIST, tr_body, 0)

        descs = []
        for j in range(NG):
            lo = j * GI
            sz = min(GI, NROW - lo)
            descs.append(pltpu.async_copy(
                itab_hbm.at[idx_v.at[buf, pl.ds(lo, sz)]],
                rows_v.at[buf, pl.ds(lo, sz)], gsem))
        return descs

    wv = [w_v[pl.ds(k, 16)] for k in (0, 16, 32, 48)]
    bias = w_v[pl.ds(56, 16)][0]

    def compute(ch, buf):
        r0 = base + ch * C

        def row_body(c, _):
            z = jnp.full((16,), bias, dtype=jnp.float32)
            a0 = z
            a1 = z
            for n in range(N_HIST):
                w = wv[n // 16][n % 16]
                fr = n * C + c
                r0v = rows_v[buf, fr, pl.ds(0, 16)]
                r1v = rows_v[buf, fr, pl.ds(16, 16)]
                a0 = a0 + w * r0v
                a1 = a1 + w * r1v
            u0 = urows_v[buf, c, pl.ds(0, 16)]
            u1 = urows_v[buf, c, pl.ds(16, 16)]
            out_v[buf, c, pl.ds(0, 16)] = u0
            out_v[buf, c, pl.ds(16, 16)] = u1
            out_v[buf, c, pl.ds(32, 16)] = u0 * a0
            out_v[buf, c, pl.ds(48, 16)] = u1 * a1
            out_v[buf, c, pl.ds(64, 16)] = a0
            out_v[buf, c, pl.ds(80, 16)] = a1
            return 0

        lax.fori_loop(0, C, row_body, 0)
        return pltpu.async_copy(
            out_v.at[buf], out_hbm.at[pl.ds(r0, C)], osem)

    gsems = (gsemA, gsemB)
    pending = stage(0, 0, gsems[0])
    out_descs = []
    for ch in range(NCH):
        buf = ch % 2
        nxt = None
        if ch + 1 < NCH:
            nxt = stage(ch + 1, 1 - buf, gsems[1 - buf])
        for dsc in pending:
            dsc.wait()
        pending = nxt
        if ch >= 2:
            out_descs[ch - 2].wait()
        out_descs.append(compute(ch, buf))
    out_descs[-2].wait()
    out_descs[-1].wait()


@jax.jit
def _run(u_rows, memory, item_table, conv_w, conv_b):
    mesh = plsc.VectorSubcoreMesh(
        core_axis_name="c", subcore_axis_name="s", num_cores=2, num_subcores=16)
    f = pl.kernel(
        _body,
        out_type=jax.ShapeDtypeStruct((BATCH, 96), jnp.float32),
        mesh=mesh,
        scratch_types=[
            pltpu.VMEM((2, C * N_HIST), jnp.int32),
            pltpu.VMEM((2, NROW), jnp.int32),
            pltpu.VMEM((2, NROW, D), jnp.float32),
            pltpu.VMEM((2, C, D), jnp.float32),
            pltpu.VMEM((2, C, 96), jnp.float32),
            pltpu.VMEM((80,), jnp.float32),
            pltpu.SemaphoreType.DMA,
            pltpu.SemaphoreType.DMA,
            pltpu.SemaphoreType.DMA,
        ],
        compiler_params=pltpu.CompilerParams(use_tc_tiling_on_sc=False, needs_layout_passes=False),
    )
    return f(u_rows, memory, item_table, conv_w, conv_b)


def kernel(user, memory, user_table, item_table, conv_w, conv_b):
    u_rows = jnp.take(user_table, user, axis=0)
    return _run(u_rows, memory.reshape(-1), item_table, conv_w, conv_b)
